# baseline (speedup 1.0000x reference)
import functools

import jax
import jax.numpy as jnp
from jax import lax
from jax.experimental import pallas as pl
from jax.experimental.pallas import tpu as pltpu
from jax.experimental.pallas import tpu_sc as plsc

N = 16384
IN_DIM = 768
HID = 64
K = 1024
TOP_M = 512
BT = 1024
GRID = N // BT
CAPC = 1024

NTILES = 16
TPT = N // NTILES
CAP = TPT
GCAP = 2048
OIDX_CAP = TOP_M + NTILES
OCOD_CAP = 3 * TOP_M + 3 * NTILES
DUMP_RANK = 1 << 20


def _dense_body(h0_ref, w1_ref, b1_ref, w2_ref, b2_ref,
                wm_ref, bm_ref, wt_ref, bt_ref, wp_ref, bp_ref,
                cbm_ref, cbt_ref, cbp_ref, kw_ref, kb_ref,
                k0_ref, k1_ref, k2_ref, alpha_ref, thr_ref,
                cn_ref, abits_ref):
    i = pl.program_id(0)

    @pl.when(i == 0)
    def _init():
        for c, cb_ref in enumerate((cbm_ref, cbt_ref, cbp_ref)):
            cb = cb_ref[...]
            cn_ref[pl.ds(c * K, K), :] = cb / (
                jnp.sqrt(jnp.sum(cb * cb, axis=-1, keepdims=True)) + 1e-8)

    x = h0_ref[...]
    h1 = jax.nn.relu(jnp.dot(x, w1_ref[...]) + b1_ref[...])
    ht = jax.nn.relu(
        jax.lax.dot_general(w2_ref[...], h1, (((0,), (1,)), ((), ())))
        + b2_ref[...])
    idxk = jax.lax.broadcasted_iota(jnp.int32, (K, 8), 0)
    colk = jax.lax.broadcasted_iota(jnp.int32, (K, 8), 1)
    sqk = idxk * idxk
    wmat = jnp.where(
        colk == 0, idxk >> 2,
        jnp.where(colk == 1, idxk & 3,
                  jnp.where(colk == 2, 1,
                            jnp.where(colk == 3, sqk >> 16,
                                      jnp.where(colk == 4, (sqk >> 8) & 255,
                                                jnp.where(colk == 5, sqk & 255,
                                                          0)))))).astype(
                                                              jnp.float32)
    for c, (w_ref, b_ref, cb_ref) in enumerate((
            (wm_ref, bm_ref, cbm_ref), (wt_ref, bt_ref, cbt_ref),
            (wp_ref, bp_ref, cbp_ref))):
        zt = jax.lax.dot_general(
            w_ref[...], ht, (((0,), (0,)), ((), ()))) + b_ref[...]
        znt = zt / (jnp.sqrt(jnp.sum(zt * zt, axis=0, keepdims=True)) + 1e-8)
        dist = -jax.lax.dot_general(
            cn_ref[pl.ds(c * K, K), :], znt, (((1,), (0,)), ((), ())))
        m = jnp.min(dist, axis=0, keepdims=True)
        eq = (dist == m).astype(jnp.float32)
        sums = jax.lax.dot_general(wmat, eq, (((0,), (0,)), ((), ())))
        sa = 4.0 * sums[0:1, :] + sums[1:2, :]
        cnt = sums[2:3, :]
        sq2 = 65536.0 * sums[3:4, :] + 256.0 * sums[4:5, :] + sums[5:6, :]
        tie2 = (sa - jnp.sqrt(jnp.maximum(2.0 * sq2 - sa * sa, 0.0))) * 0.5
        idxf = jnp.where(cnt > 1.5, tie2, sa)
        (k0_ref, k1_ref, k2_ref)[c][...] = idxf.astype(jnp.int32).reshape(BT)
    trow = jax.lax.dot_general(
        kw_ref[...], ht, (((0,), (0,)), ((), ()))) + kb_ref[...]
    asig = jax.nn.sigmoid(trow)
    alpha_ref[...] = asig.reshape(BT)
    abits_ref[pl.ds(i, 1), :] = jax.lax.bitcast_convert_type(asig, jnp.int32)

    @pl.when(i == GRID - 1)
    def _threshold():
        allbits = abits_ref[...]

        def cond(st):
            lo, hi, clo = st
            return jnp.logical_and(clo > CAPC, hi - lo > 1)

        def body(st):
            lo, hi, clo = st
            mid = lo + (hi - lo) // 2
            c = jnp.sum((allbits >= mid).astype(jnp.int32))
            big = c >= TOP_M
            return (jnp.where(big, mid, lo), jnp.where(big, hi, mid),
                    jnp.where(big, c, clo))

        lo, _, _ = lax.while_loop(
            cond, body,
            (jnp.int32(0), jnp.int32(0x7F800000), jnp.int32(N)))
        thr_ref[...] = jnp.full(
            (1, 128), jax.lax.bitcast_convert_type(lo, jnp.float32),
            jnp.float32)


@functools.partial(jax.jit, static_argnames=("interpret",))
def _dense_call(h0, enc_W1, enc_b1, enc_W2, enc_b2, Wm_W, Wm_b, Wt_W, Wt_b,
                Wp_W, Wp_b, cb_m, cb_t, cb_p, key_W, key_b, interpret=False):
    full2 = lambda r, cdim: pl.BlockSpec((r, cdim), lambda i: (0, 0))
    in_specs = [
        pl.BlockSpec((BT, IN_DIM), lambda i: (i, 0)),
        full2(IN_DIM, HID), full2(1, HID),
        full2(HID, HID), full2(HID, 1),
        full2(HID, HID), full2(HID, 1),
        full2(HID, HID), full2(HID, 1),
        full2(HID, HID), full2(HID, 1),
        full2(K, HID), full2(K, HID), full2(K, HID),
        full2(HID, 1), full2(1, 1),
    ]
    out_specs = [
        pl.BlockSpec((BT,), lambda i: (i,)),
        pl.BlockSpec((BT,), lambda i: (i,)),
        pl.BlockSpec((BT,), lambda i: (i,)),
        pl.BlockSpec((BT,), lambda i: (i,)),
        pl.BlockSpec((1, 128), lambda i: (0, 0)),
    ]
    out_shape = [
        jax.ShapeDtypeStruct((N,), jnp.int32),
        jax.ShapeDtypeStruct((N,), jnp.int32),
        jax.ShapeDtypeStruct((N,), jnp.int32),
        jax.ShapeDtypeStruct((N,), jnp.float32),
        jax.ShapeDtypeStruct((1, 128), jnp.float32),
    ]
    k0, k1, k2, alpha, thr = pl.pallas_call(
        _dense_body,
        grid=(GRID,),
        in_specs=in_specs,
        out_specs=out_specs,
        out_shape=out_shape,
        scratch_shapes=[pltpu.VMEM((3 * K, HID), jnp.float32),
                        pltpu.VMEM((GRID, BT), jnp.int32)],
        interpret=interpret,
    )(h0, enc_W1, enc_b1.reshape(1, HID), enc_W2, enc_b2.reshape(HID, 1),
      Wm_W, Wm_b.reshape(HID, 1), Wt_W, Wt_b.reshape(HID, 1),
      Wp_W, Wp_b.reshape(HID, 1),
      cb_m, cb_t, cb_p, key_W, key_b.reshape(1, 1))
    return k0, k1, k2, alpha, thr


def _topk_body(alpha_hbm, thr_hbm, km_hbm, kt_hbm, kp_hbm,
               outidx_hbm, outcodes_hbm,
               alpha_v, cod0_v, cod1_v, cod2_v, cv, ci, cr, thr_v, cnts_v,
               gv, gi, meta_v, sidx_v, cbuf_v, obuf32_v, obuf96_v, dsem,
               sh_cnts, sh_v, sh_i, sh_oi, sh_oc):
    cid = lax.axis_index("c")
    sid = lax.axis_index("s")
    lane = lax.iota(jnp.int32, 16)
    base = pl.multiple_of(sid * TPT, TPT)

    @pl.when(cid == 0)
    def _phase1():
        pltpu.sync_copy(alpha_hbm.at[pl.ds(base, TPT)], alpha_v)
        pltpu.sync_copy(thr_hbm.at[pl.ds(0, 16)], thr_v)
        for chbm, cod_v in ((km_hbm, cod0_v), (kt_hbm, cod1_v),
                            (kp_hbm, cod2_v)):
            pltpu.sync_copy(chbm.at[pl.ds(base, TPT)], cod_v)
        thr = thr_v[...]
        neg1 = jnp.full((16,), -1.0, jnp.float32)
        basev = jnp.full((16,), base, jnp.int32)
        dumpv = jnp.full((16,), DUMP_RANK, jnp.int32)
        for j in range(CAP // 16):
            cv[pl.ds(16 * j, 16)] = neg1
            ci[pl.ds(16 * j, 16)] = basev
            cr[pl.ds(16 * j, 16)] = dumpv

        @pl.when(sid == 0)
        def _prefill_shared():
            pltpu.sync_copy(cv, sh_v.at[pl.ds(0, CAP)])
            pltpu.sync_copy(cv, sh_v.at[pl.ds(CAP, CAP)])

        def sel(j, cnt):
            a = plsc.load_gather(alpha_v, [16 * j + lane])
            msk = a >= thr
            inc = msk.astype(jnp.int32)
            pc = plsc.cumsum(inc)
            tot = jnp.sum(inc, axis=0)
            pos = pc - 1 + cnt
            plsc.store_scatter(cv, [pos], a, mask=msk)
            plsc.store_scatter(ci, [pos], basev + 16 * j + lane, mask=msk)
            return cnt + tot

        cnt = lax.fori_loop(0, TPT // 16, sel, jnp.int32(0))
        cnt_pad = (cnt + 15) & (-16)
        meta_v[pl.ds(0, 16)] = jnp.full((16,), cnt_pad, jnp.int32)
        pltpu.sync_copy(meta_v.at[pl.ds(0, 16)],
                        sh_cnts.at[pl.ds(pl.multiple_of(16 * sid, 16), 16)])

    plsc.subcore_barrier()

    @pl.when(cid == 0)
    def _phase2():
        pltpu.sync_copy(sh_cnts, cnts_v)
        cnt_pad = meta_v[pl.ds(0, 16)][0]
        off = jnp.int32(0)
        total = jnp.int32(0)
        for w in range(NTILES):
            cw = cnts_v[pl.ds(16 * w, 16)][0]
            off = off + jnp.where(w < sid, cw, 0)
            total = total + cw
        meta_v[pl.ds(16, 16)] = jnp.full((16,), off, jnp.int32)
        meta_v[pl.ds(32, 16)] = jnp.full((16,), total, jnp.int32)

        def cp(t, carry):
            s = pl.multiple_of(16 * t, 16)
            d = pl.multiple_of(off + 16 * t, 16)
            pltpu.sync_copy(cv.at[pl.ds(s, 16)], sh_v.at[pl.ds(d, 16)])
            pltpu.sync_copy(ci.at[pl.ds(s, 16)], sh_i.at[pl.ds(d, 16)])
            return carry

        lax.fori_loop(0, cnt_pad // 16, cp, jnp.int32(0))

    plsc.subcore_barrier()

    @pl.when(cid == 0)
    def _phase3():
        cnt_pad = meta_v[pl.ds(0, 16)][0]
        total = meta_v[pl.ds(32, 16)][0]
        pltpu.sync_copy(sh_v, gv)
        pltpu.sync_copy(sh_i, gi)
        ng = total // 16

        def own_loop(o, carry):
            ov = plsc.load_gather(cv, [16 * o + lane])
            oi = plsc.load_gather(ci, [16 * o + lane])
            rankvec = jnp.zeros((16,), jnp.int32)
            for half in range(2):
                vs = [jnp.full((16,), ov[8 * half + j], jnp.float32)
                      for j in range(8)]
                ts = [jnp.full((16,), oi[8 * half + j], jnp.int32)
                      for j in range(8)]

                def g_loop(g, accs):
                    gv16 = plsc.load_gather(gv, [16 * g + lane])
                    gi16 = plsc.load_gather(gi, [16 * g + lane])
                    return tuple(
                        accs[j] + jnp.logical_or(
                            gv16 > vs[j],
                            jnp.logical_and(gv16 == vs[j], gi16 < ts[j])
                        ).astype(jnp.int32)
                        for j in range(8))

                accs = lax.fori_loop(
                    0, ng, g_loop,
                    tuple(jnp.zeros((16,), jnp.int32) for _ in range(8)))
                for j in range(8):
                    rj = jnp.sum(accs[j], axis=0)
                    rankvec = jnp.where(lane == 8 * half + j, rj, rankvec)
            plsc.store_scatter(cr, [16 * o + lane], rankvec)
            return carry

        lax.fori_loop(0, cnt_pad // 16, own_loop, jnp.int32(0))

        def emit(t, carry):
            r16 = plsc.load_gather(cr, [16 * t + lane])
            t4 = jnp.full((16,), 4 * t, jnp.int32)
            ridx = jnp.where(r16 < TOP_M, r16, TOP_M + sid)
            plsc.store_scatter(sidx_v, [t4, lane], ridx)
            loc16 = plsc.load_gather(ci, [16 * t + lane]) - base
            waits = [pltpu.async_copy(
                ci.at[pl.ds(pl.multiple_of(16 * t, 16), 16)],
                sh_oi.at[sidx_v.at[4 * t]], dsem)]
            for c, cod_v in enumerate((cod0_v, cod1_v, cod2_v)):
                cc = plsc.load_gather(cod_v, [loc16])
                cbuf_v[pl.ds(16 * c, 16)] = cc
                cidx = jnp.where(r16 < TOP_M, 3 * r16 + c,
                                 3 * TOP_M + 3 * sid + c)
                plsc.store_scatter(sidx_v, [t4 + 1 + c, lane], cidx)
                waits.append(pltpu.async_copy(
                    cbuf_v.at[pl.ds(16 * c, 16)],
                    sh_oc.at[sidx_v.at[4 * t + 1 + c]], dsem))
            for wgo in waits:
                wgo.wait()
            return carry

        lax.fori_loop(0, cnt_pad // 16, emit, jnp.int32(0))

    plsc.subcore_barrier()

    @pl.when(cid == 0)
    def _phase4():
        oi_off = pl.multiple_of(32 * sid, 32)
        oc_off = pl.multiple_of(96 * sid, 8)
        pltpu.sync_copy(sh_oi.at[pl.ds(oi_off, 32)], obuf32_v)
        pltpu.sync_copy(obuf32_v, outidx_hbm.at[pl.ds(oi_off, 32)])
        pltpu.sync_copy(sh_oc.at[pl.ds(oc_off, 96)], obuf96_v)
        pltpu.sync_copy(obuf96_v, outcodes_hbm.at[pl.ds(oc_off, 96)])


_topk_sc = functools.partial(
    pl.kernel,
    out_type=[jax.ShapeDtypeStruct((TOP_M,), jnp.int32),
              jax.ShapeDtypeStruct((3 * TOP_M,), jnp.int32)],
    mesh=plsc.VectorSubcoreMesh(core_axis_name="c", subcore_axis_name="s"),
    compiler_params=pltpu.CompilerParams(needs_layout_passes=False),
    scratch_types=[
        pltpu.VMEM((TPT,), jnp.float32),
        pltpu.VMEM((TPT,), jnp.int32),
        pltpu.VMEM((TPT,), jnp.int32),
        pltpu.VMEM((TPT,), jnp.int32),
        pltpu.VMEM((CAP,), jnp.float32),
        pltpu.VMEM((CAP,), jnp.int32),
        pltpu.VMEM((CAP,), jnp.int32),
        pltpu.VMEM((16,), jnp.float32),
        pltpu.VMEM((NTILES * 16,), jnp.int32),
        pltpu.VMEM((GCAP,), jnp.float32),
        pltpu.VMEM((GCAP,), jnp.int32),
        pltpu.VMEM((48,), jnp.int32),
        pltpu.VMEM((4 * (CAP // 16), 16), jnp.int32),
        pltpu.VMEM((48,), jnp.int32),
        pltpu.VMEM((32,), jnp.int32),
        pltpu.VMEM((96,), jnp.int32),
        pltpu.SemaphoreType.DMA,
        pltpu.VMEM_SHARED((NTILES * 16,), jnp.int32),
        pltpu.VMEM_SHARED((GCAP,), jnp.float32),
        pltpu.VMEM_SHARED((GCAP,), jnp.int32),
        pltpu.VMEM_SHARED((OIDX_CAP,), jnp.int32),
        pltpu.VMEM_SHARED((OCOD_CAP,), jnp.int32),
    ])(_topk_body)


def kernel(h0, enc_W1, enc_b1, enc_W2, enc_b2, Wm_W, Wm_b, Wt_W, Wt_b,
           Wp_W, Wp_b, cb_m, cb_t, cb_p, key_W, key_b):
    k0, k1, k2, alpha, thr = _dense_call(
        h0, enc_W1, enc_b1, enc_W2, enc_b2, Wm_W, Wm_b, Wt_W, Wt_b,
        Wp_W, Wp_b, cb_m, cb_t, cb_p, key_W, key_b)
    key_idx, sel_codes = _topk_sc(alpha, thr.reshape(-1), k0, k1, k2)
    codes = jnp.stack((k0, k1, k2), axis=-1)
    return (codes, key_idx, sel_codes.reshape(TOP_M, 3))


# speedup vs baseline: 1.0029x; 1.0029x over previous
import functools

import jax
import jax.numpy as jnp
from jax import lax
from jax.experimental import pallas as pl
from jax.experimental.pallas import tpu as pltpu
from jax.experimental.pallas import tpu_sc as plsc

N = 16384
IN_DIM = 768
HID = 64
K = 1024
TOP_M = 512
BT = 1024
GRID = N // BT
CAPC = 1024

NTILES = 16
TPT = N // NTILES
CAP = TPT
GCAP = 2048
OIDX_CAP = TOP_M + NTILES
OCOD_CAP = 3 * TOP_M + 3 * NTILES
DUMP_RANK = 1 << 20


def _dense_body(h0_ref, w1_ref, b1_ref, w2_ref, b2_ref,
                wm_ref, bm_ref, wt_ref, bt_ref, wp_ref, bp_ref,
                cbm_ref, cbt_ref, cbp_ref, kw_ref, kb_ref,
                k0_ref, k1_ref, k2_ref, alpha_ref, thr_ref,
                cn_ref, abits_ref):
    i = pl.program_id(0)

    @pl.when(i == 0)
    def _init():
        for c, cb_ref in enumerate((cbm_ref, cbt_ref, cbp_ref)):
            cb = cb_ref[...]
            cn_ref[pl.ds(c * K, K), :] = cb / (
                jnp.sqrt(jnp.sum(cb * cb, axis=-1, keepdims=True)) + 1e-8)

    x = h0_ref[...]
    h1 = jax.nn.relu(jnp.dot(x, w1_ref[...]) + b1_ref[...])
    ht = jax.nn.relu(
        jax.lax.dot_general(w2_ref[...], h1, (((0,), (1,)), ((), ())))
        + b2_ref[...])
    idxk = jax.lax.broadcasted_iota(jnp.int32, (K, 8), 0)
    colk = jax.lax.broadcasted_iota(jnp.int32, (K, 8), 1)
    sqk = idxk * idxk
    wmat = jnp.where(
        colk == 0, idxk >> 2,
        jnp.where(colk == 1, idxk & 3,
                  jnp.where(colk == 2, 1,
                            jnp.where(colk == 3, sqk >> 16,
                                      jnp.where(colk == 4, (sqk >> 8) & 255,
                                                jnp.where(colk == 5, sqk & 255,
                                                          0)))))).astype(
                                                              jnp.float32)
    for c, (w_ref, b_ref, cb_ref) in enumerate((
            (wm_ref, bm_ref, cbm_ref), (wt_ref, bt_ref, cbt_ref),
            (wp_ref, bp_ref, cbp_ref))):
        zt = jax.lax.dot_general(
            w_ref[...], ht, (((0,), (0,)), ((), ()))) + b_ref[...]
        znt = zt / (jnp.sqrt(jnp.sum(zt * zt, axis=0, keepdims=True)) + 1e-8)
        dist = -jax.lax.dot_general(
            cn_ref[pl.ds(c * K, K), :], znt, (((1,), (0,)), ((), ())))
        m = jnp.min(dist, axis=0, keepdims=True)
        eq = (dist == m).astype(jnp.float32)
        sums = jax.lax.dot_general(wmat, eq, (((0,), (0,)), ((), ())))
        sa = 4.0 * sums[0:1, :] + sums[1:2, :]
        cnt = sums[2:3, :]
        sq2 = 65536.0 * sums[3:4, :] + 256.0 * sums[4:5, :] + sums[5:6, :]
        tie2 = (sa - jnp.sqrt(jnp.maximum(2.0 * sq2 - sa * sa, 0.0))) * 0.5
        idxf = jnp.where(cnt > 1.5, tie2, sa)
        (k0_ref, k1_ref, k2_ref)[c][...] = idxf.astype(jnp.int32).reshape(BT)
    trow = jax.lax.dot_general(
        kw_ref[...], ht, (((0,), (0,)), ((), ()))) + kb_ref[...]
    asig = jax.nn.sigmoid(trow)
    alpha_ref[...] = asig.reshape(BT)
    abits_ref[pl.ds(i, 1), :] = jax.lax.bitcast_convert_type(asig, jnp.int32)

    @pl.when(i == GRID - 1)
    def _threshold():
        allbits = abits_ref[...]

        def cond(st):
            lo, hi, clo = st
            return jnp.logical_and(clo > CAPC, hi - lo > 1)

        def body(st):
            lo, hi, clo = st
            mid = lo + (hi - lo) // 2
            c = jnp.sum((allbits >= mid).astype(jnp.int32))
            big = c >= TOP_M
            return (jnp.where(big, mid, lo), jnp.where(big, hi, mid),
                    jnp.where(big, c, clo))

        lo, _, _ = lax.while_loop(
            cond, body,
            (jnp.int32(0), jnp.int32(0x7F800000), jnp.int32(N)))
        thr_ref[...] = jnp.full(
            (1, 128), jax.lax.bitcast_convert_type(lo, jnp.float32),
            jnp.float32)


@functools.partial(jax.jit, static_argnames=("interpret",))
def _dense_call(h0, enc_W1, enc_b1, enc_W2, enc_b2, Wm_W, Wm_b, Wt_W, Wt_b,
                Wp_W, Wp_b, cb_m, cb_t, cb_p, key_W, key_b, interpret=False):
    full2 = lambda r, cdim: pl.BlockSpec((r, cdim), lambda i: (0, 0))
    in_specs = [
        pl.BlockSpec((BT, IN_DIM), lambda i: (i, 0)),
        full2(IN_DIM, HID), full2(1, HID),
        full2(HID, HID), full2(HID, 1),
        full2(HID, HID), full2(HID, 1),
        full2(HID, HID), full2(HID, 1),
        full2(HID, HID), full2(HID, 1),
        full2(K, HID), full2(K, HID), full2(K, HID),
        full2(HID, 1), full2(1, 1),
    ]
    out_specs = [
        pl.BlockSpec((BT,), lambda i: (i,)),
        pl.BlockSpec((BT,), lambda i: (i,)),
        pl.BlockSpec((BT,), lambda i: (i,)),
        pl.BlockSpec((BT,), lambda i: (i,)),
        pl.BlockSpec((1, 128), lambda i: (0, 0)),
    ]
    out_shape = [
        jax.ShapeDtypeStruct((N,), jnp.int32),
        jax.ShapeDtypeStruct((N,), jnp.int32),
        jax.ShapeDtypeStruct((N,), jnp.int32),
        jax.ShapeDtypeStruct((N,), jnp.float32),
        jax.ShapeDtypeStruct((1, 128), jnp.float32),
    ]
    k0, k1, k2, alpha, thr = pl.pallas_call(
        _dense_body,
        grid=(GRID,),
        in_specs=in_specs,
        out_specs=out_specs,
        out_shape=out_shape,
        scratch_shapes=[pltpu.VMEM((3 * K, HID), jnp.float32),
                        pltpu.VMEM((GRID, BT), jnp.int32)],
        interpret=interpret,
    )(h0, enc_W1, enc_b1.reshape(1, HID), enc_W2, enc_b2.reshape(HID, 1),
      Wm_W, Wm_b.reshape(HID, 1), Wt_W, Wt_b.reshape(HID, 1),
      Wp_W, Wp_b.reshape(HID, 1),
      cb_m, cb_t, cb_p, key_W, key_b.reshape(1, 1))
    return k0, k1, k2, alpha, thr


def _topk_body(alpha_hbm, thr_hbm, km_hbm, kt_hbm, kp_hbm,
               outidx_hbm, outc0_hbm, outc1_hbm, outc2_hbm,
               alpha_v, cod0_v, cod1_v, cod2_v, cv, ci, cr, thr_v, cnts_v,
               gv, gi, meta_v, sidx_v, obuf32_v, dsem,
               sh_cnts, sh_v, sh_i, sh_oi):
    cid = lax.axis_index("c")
    sid = lax.axis_index("s")
    lane = lax.iota(jnp.int32, 16)
    base = pl.multiple_of(sid * TPT, TPT)

    @pl.when(cid == 0)
    def _phase1():
        pltpu.sync_copy(alpha_hbm.at[pl.ds(base, TPT)], alpha_v)
        pltpu.sync_copy(thr_hbm.at[pl.ds(0, 16)], thr_v)
        thr = thr_v[...]
        neg1 = jnp.full((16,), -1.0, jnp.float32)
        basev = jnp.full((16,), base, jnp.int32)
        dumpv = jnp.full((16,), DUMP_RANK, jnp.int32)
        for j in range(CAP // 16):
            cv[pl.ds(16 * j, 16)] = neg1
            ci[pl.ds(16 * j, 16)] = basev
            cr[pl.ds(16 * j, 16)] = dumpv

        @pl.when(sid == 0)
        def _prefill_shared():
            pltpu.sync_copy(cv, sh_v.at[pl.ds(0, CAP)])
            pltpu.sync_copy(cv, sh_v.at[pl.ds(CAP, CAP)])

        def sel(j, cnt):
            a = plsc.load_gather(alpha_v, [16 * j + lane])
            msk = a >= thr
            inc = msk.astype(jnp.int32)
            pc = plsc.cumsum(inc)
            tot = jnp.sum(inc, axis=0)
            pos = pc - 1 + cnt
            plsc.store_scatter(cv, [pos], a, mask=msk)
            plsc.store_scatter(ci, [pos], basev + 16 * j + lane, mask=msk)
            return cnt + tot

        cnt = lax.fori_loop(0, TPT // 16, sel, jnp.int32(0))
        cnt_pad = (cnt + 15) & (-16)
        meta_v[pl.ds(0, 16)] = jnp.full((16,), cnt_pad, jnp.int32)
        pltpu.sync_copy(meta_v.at[pl.ds(0, 16)],
                        sh_cnts.at[pl.ds(pl.multiple_of(16 * sid, 16), 16)])

    plsc.subcore_barrier()

    @pl.when(cid == 0)
    def _phase2():
        pltpu.sync_copy(sh_cnts, cnts_v)
        cnt_pad = meta_v[pl.ds(0, 16)][0]
        off = jnp.int32(0)
        total = jnp.int32(0)
        for w in range(NTILES):
            cw = cnts_v[pl.ds(16 * w, 16)][0]
            off = off + jnp.where(w < sid, cw, 0)
            total = total + cw
        meta_v[pl.ds(16, 16)] = jnp.full((16,), off, jnp.int32)
        meta_v[pl.ds(32, 16)] = jnp.full((16,), total, jnp.int32)

        def cp(t, carry):
            s = pl.multiple_of(16 * t, 16)
            d = pl.multiple_of(off + 16 * t, 16)
            pltpu.sync_copy(cv.at[pl.ds(s, 16)], sh_v.at[pl.ds(d, 16)])
            pltpu.sync_copy(ci.at[pl.ds(s, 16)], sh_i.at[pl.ds(d, 16)])
            return carry

        lax.fori_loop(0, cnt_pad // 16, cp, jnp.int32(0))

    plsc.subcore_barrier()

    @pl.when(cid == 0)
    def _phase3():
        cnt_pad = meta_v[pl.ds(0, 16)][0]
        total = meta_v[pl.ds(32, 16)][0]
        pltpu.sync_copy(sh_v, gv)
        pltpu.sync_copy(sh_i, gi)
        ng = total // 16

        def own_loop(o, carry):
            ov = plsc.load_gather(cv, [16 * o + lane])
            oi = plsc.load_gather(ci, [16 * o + lane])
            rankvec = jnp.zeros((16,), jnp.int32)
            for half in range(2):
                vs = [jnp.full((16,), ov[8 * half + j], jnp.float32)
                      for j in range(8)]
                ts = [jnp.full((16,), oi[8 * half + j], jnp.int32)
                      for j in range(8)]

                def g_loop(g, accs):
                    gv16 = plsc.load_gather(gv, [16 * g + lane])
                    gi16 = plsc.load_gather(gi, [16 * g + lane])
                    return tuple(
                        accs[j] + jnp.logical_or(
                            gv16 > vs[j],
                            jnp.logical_and(gv16 == vs[j], gi16 < ts[j])
                        ).astype(jnp.int32)
                        for j in range(8))

                accs = lax.fori_loop(
                    0, ng, g_loop,
                    tuple(jnp.zeros((16,), jnp.int32) for _ in range(8)))
                for j in range(8):
                    rj = jnp.sum(accs[j], axis=0)
                    rankvec = jnp.where(lane == 8 * half + j, rj, rankvec)
            plsc.store_scatter(cr, [16 * o + lane], rankvec)
            return carry

        lax.fori_loop(0, cnt_pad // 16, own_loop, jnp.int32(0))

        def emit(t, carry):
            r16 = plsc.load_gather(cr, [16 * t + lane])
            tsplat = jnp.full((16,), t, jnp.int32)
            ridx = jnp.where(r16 < TOP_M, r16, TOP_M + sid)
            plsc.store_scatter(sidx_v, [tsplat, lane], ridx)
            pltpu.async_copy(
                ci.at[pl.ds(pl.multiple_of(16 * t, 16), 16)],
                sh_oi.at[sidx_v.at[t]], dsem).wait()
            return carry

        lax.fori_loop(0, cnt_pad // 16, emit, jnp.int32(0))

    plsc.subcore_barrier()

    @pl.when(cid == 0)
    def _phase4():
        oi_off = pl.multiple_of(32 * sid, 32)
        pltpu.sync_copy(sh_oi.at[pl.ds(oi_off, 32)], obuf32_v)
        w0 = pltpu.async_copy(obuf32_v, outidx_hbm.at[pl.ds(oi_off, 32)],
                              dsem)
        waits = [w0]
        for chbm, cod_v, out_hbm in (
                (km_hbm, cod0_v, outc0_hbm), (kt_hbm, cod1_v, outc1_hbm),
                (kp_hbm, cod2_v, outc2_hbm)):
            waits.append(pltpu.async_copy(chbm.at[obuf32_v], cod_v, dsem))
        for wgo in waits:
            wgo.wait()
        for cod_v, out_hbm in ((cod0_v, outc0_hbm), (cod1_v, outc1_hbm),
                               (cod2_v, outc2_hbm)):
            pltpu.sync_copy(cod_v, out_hbm.at[pl.ds(oi_off, 32)])


_topk_sc = functools.partial(
    pl.kernel,
    out_type=[jax.ShapeDtypeStruct((TOP_M,), jnp.int32),
              jax.ShapeDtypeStruct((TOP_M,), jnp.int32),
              jax.ShapeDtypeStruct((TOP_M,), jnp.int32),
              jax.ShapeDtypeStruct((TOP_M,), jnp.int32)],
    mesh=plsc.VectorSubcoreMesh(core_axis_name="c", subcore_axis_name="s"),
    compiler_params=pltpu.CompilerParams(needs_layout_passes=False),
    scratch_types=[
        pltpu.VMEM((TPT,), jnp.float32),
        pltpu.VMEM((32,), jnp.int32),
        pltpu.VMEM((32,), jnp.int32),
        pltpu.VMEM((32,), jnp.int32),
        pltpu.VMEM((CAP,), jnp.float32),
        pltpu.VMEM((CAP,), jnp.int32),
        pltpu.VMEM((CAP,), jnp.int32),
        pltpu.VMEM((16,), jnp.float32),
        pltpu.VMEM((NTILES * 16,), jnp.int32),
        pltpu.VMEM((GCAP,), jnp.float32),
        pltpu.VMEM((GCAP,), jnp.int32),
        pltpu.VMEM((48,), jnp.int32),
        pltpu.VMEM((CAP // 16, 16), jnp.int32),
        pltpu.VMEM((32,), jnp.int32),
        pltpu.SemaphoreType.DMA,
        pltpu.VMEM_SHARED((NTILES * 16,), jnp.int32),
        pltpu.VMEM_SHARED((GCAP,), jnp.float32),
        pltpu.VMEM_SHARED((GCAP,), jnp.int32),
        pltpu.VMEM_SHARED((OIDX_CAP,), jnp.int32),
    ])(_topk_body)


def kernel(h0, enc_W1, enc_b1, enc_W2, enc_b2, Wm_W, Wm_b, Wt_W, Wt_b,
           Wp_W, Wp_b, cb_m, cb_t, cb_p, key_W, key_b):
    k0, k1, k2, alpha, thr = _dense_call(
        h0, enc_W1, enc_b1, enc_W2, enc_b2, Wm_W, Wm_b, Wt_W, Wt_b,
        Wp_W, Wp_b, cb_m, cb_t, cb_p, key_W, key_b)
    key_idx, s0, s1, s2 = _topk_sc(alpha, thr.reshape(-1), k0, k1, k2)
    codes = jnp.stack((k0, k1, k2), axis=-1)
    return (codes, key_idx, jnp.stack((s0, s1, s2), axis=-1))


# speedup vs baseline: 1.0974x; 1.0942x over previous
import functools

import jax
import jax.numpy as jnp
from jax import lax
from jax.experimental import pallas as pl
from jax.experimental.pallas import tpu as pltpu
from jax.experimental.pallas import tpu_sc as plsc

N = 16384
IN_DIM = 768
HID = 64
K = 1024
TOP_M = 512
BT = 2048
GRID = N // BT
CAPC = 1024

NTILES = 16
TPT = N // NTILES
CAP = TPT
GCAP = 2048
OIDX_CAP = TOP_M + NTILES
OCOD_CAP = 3 * TOP_M + 3 * NTILES
DUMP_RANK = 1 << 20


def _dense_body(h0_ref, w1_ref, b1_ref, w2_ref, b2_ref,
                wm_ref, bm_ref, wt_ref, bt_ref, wp_ref, bp_ref,
                cbm_ref, cbt_ref, cbp_ref, kw_ref, kb_ref,
                k0_ref, k1_ref, k2_ref, alpha_ref, thr_ref,
                cn_ref, abits_ref):
    i = pl.program_id(0)

    @pl.when(i == 0)
    def _init():
        for c, cb_ref in enumerate((cbm_ref, cbt_ref, cbp_ref)):
            cb = cb_ref[...]
            cn_ref[pl.ds(c * K, K), :] = cb / (
                jnp.sqrt(jnp.sum(cb * cb, axis=-1, keepdims=True)) + 1e-8)

    x = h0_ref[...]
    h1 = jax.nn.relu(jnp.dot(x, w1_ref[...]) + b1_ref[...])
    ht = jax.nn.relu(
        jax.lax.dot_general(w2_ref[...], h1, (((0,), (1,)), ((), ())))
        + b2_ref[...])
    idxk = jax.lax.broadcasted_iota(jnp.int32, (K, 8), 0)
    colk = jax.lax.broadcasted_iota(jnp.int32, (K, 8), 1)
    sqk = idxk * idxk
    wmat = jnp.where(
        colk == 0, idxk >> 2,
        jnp.where(colk == 1, idxk & 3,
                  jnp.where(colk == 2, 1,
                            jnp.where(colk == 3, sqk >> 16,
                                      jnp.where(colk == 4, (sqk >> 8) & 255,
                                                jnp.where(colk == 5, sqk & 255,
                                                          0)))))).astype(
                                                              jnp.float32)
    for c, (w_ref, b_ref, cb_ref) in enumerate((
            (wm_ref, bm_ref, cbm_ref), (wt_ref, bt_ref, cbt_ref),
            (wp_ref, bp_ref, cbp_ref))):
        zt = jax.lax.dot_general(
            w_ref[...], ht, (((0,), (0,)), ((), ()))) + b_ref[...]
        znt = zt / (jnp.sqrt(jnp.sum(zt * zt, axis=0, keepdims=True)) + 1e-8)
        dist = -jax.lax.dot_general(
            cn_ref[pl.ds(c * K, K), :], znt, (((1,), (0,)), ((), ())))
        m = jnp.min(dist, axis=0, keepdims=True)
        eq = (dist == m).astype(jnp.float32)
        sums = jax.lax.dot_general(wmat, eq, (((0,), (0,)), ((), ())))
        sa = 4.0 * sums[0:1, :] + sums[1:2, :]
        cnt = sums[2:3, :]
        sq2 = 65536.0 * sums[3:4, :] + 256.0 * sums[4:5, :] + sums[5:6, :]
        tie2 = (sa - jnp.sqrt(jnp.maximum(2.0 * sq2 - sa * sa, 0.0))) * 0.5
        idxf = jnp.where(cnt > 1.5, tie2, sa)
        (k0_ref, k1_ref, k2_ref)[c][...] = idxf.astype(jnp.int32).reshape(BT)
    trow = jax.lax.dot_general(
        kw_ref[...], ht, (((0,), (0,)), ((), ()))) + kb_ref[...]
    asig = jax.nn.sigmoid(trow)
    alpha_ref[...] = asig.reshape(BT)
    abits_ref[pl.ds(i, 1), :] = jax.lax.bitcast_convert_type(asig, jnp.int32)

    @pl.when(i == GRID - 1)
    def _threshold():
        allbits = abits_ref[...]

        def cond(st):
            lo, hi, clo = st
            return jnp.logical_and(clo > CAPC, hi - lo > 1)

        def body(st):
            lo, hi, clo = st
            mid = lo + (hi - lo) // 2
            c = jnp.sum((allbits >= mid).astype(jnp.int32))
            big = c >= TOP_M
            return (jnp.where(big, mid, lo), jnp.where(big, hi, mid),
                    jnp.where(big, c, clo))

        lo, _, _ = lax.while_loop(
            cond, body,
            (jnp.int32(0), jnp.int32(0x7F800000), jnp.int32(N)))
        thr_ref[...] = jnp.full(
            (1, 128), jax.lax.bitcast_convert_type(lo, jnp.float32),
            jnp.float32)


@functools.partial(jax.jit, static_argnames=("interpret",))
def _dense_call(h0, enc_W1, enc_b1, enc_W2, enc_b2, Wm_W, Wm_b, Wt_W, Wt_b,
                Wp_W, Wp_b, cb_m, cb_t, cb_p, key_W, key_b, interpret=False):
    full2 = lambda r, cdim: pl.BlockSpec((r, cdim), lambda i: (0, 0))
    in_specs = [
        pl.BlockSpec((BT, IN_DIM), lambda i: (i, 0)),
        full2(IN_DIM, HID), full2(1, HID),
        full2(HID, HID), full2(HID, 1),
        full2(HID, HID), full2(HID, 1),
        full2(HID, HID), full2(HID, 1),
        full2(HID, HID), full2(HID, 1),
        full2(K, HID), full2(K, HID), full2(K, HID),
        full2(HID, 1), full2(1, 1),
    ]
    out_specs = [
        pl.BlockSpec((BT,), lambda i: (i,)),
        pl.BlockSpec((BT,), lambda i: (i,)),
        pl.BlockSpec((BT,), lambda i: (i,)),
        pl.BlockSpec((BT,), lambda i: (i,)),
        pl.BlockSpec((1, 128), lambda i: (0, 0)),
    ]
    out_shape = [
        jax.ShapeDtypeStruct((N,), jnp.int32),
        jax.ShapeDtypeStruct((N,), jnp.int32),
        jax.ShapeDtypeStruct((N,), jnp.int32),
        jax.ShapeDtypeStruct((N,), jnp.float32),
        jax.ShapeDtypeStruct((1, 128), jnp.float32),
    ]
    k0, k1, k2, alpha, thr = pl.pallas_call(
        _dense_body,
        grid=(GRID,),
        in_specs=in_specs,
        out_specs=out_specs,
        out_shape=out_shape,
        scratch_shapes=[pltpu.VMEM((3 * K, HID), jnp.float32),
                        pltpu.VMEM((GRID, BT), jnp.int32)],
        interpret=interpret,
    )(h0, enc_W1, enc_b1.reshape(1, HID), enc_W2, enc_b2.reshape(HID, 1),
      Wm_W, Wm_b.reshape(HID, 1), Wt_W, Wt_b.reshape(HID, 1),
      Wp_W, Wp_b.reshape(HID, 1),
      cb_m, cb_t, cb_p, key_W, key_b.reshape(1, 1))
    return k0, k1, k2, alpha, thr


def _topk_body(alpha_hbm, thr_hbm, km_hbm, kt_hbm, kp_hbm,
               outidx_hbm, outc0_hbm, outc1_hbm, outc2_hbm,
               alpha_v, cod0_v, cod1_v, cod2_v, cv, ci, cr, thr_v, cnts_v,
               gv, gi, meta_v, sidx_v, obuf32_v, dsem,
               sh_cnts, sh_v, sh_i, sh_oi):
    cid = lax.axis_index("c")
    sid = lax.axis_index("s")
    lane = lax.iota(jnp.int32, 16)
    base = pl.multiple_of(sid * TPT, TPT)

    @pl.when(cid == 0)
    def _phase1():
        pltpu.sync_copy(alpha_hbm.at[pl.ds(base, TPT)], alpha_v)
        pltpu.sync_copy(thr_hbm.at[pl.ds(0, 16)], thr_v)
        thr = thr_v[...]
        neg1 = jnp.full((16,), -1.0, jnp.float32)
        basev = jnp.full((16,), base, jnp.int32)
        dumpv = jnp.full((16,), DUMP_RANK, jnp.int32)
        for j in range(CAP // 16):
            cv[pl.ds(16 * j, 16)] = neg1
            ci[pl.ds(16 * j, 16)] = basev
            cr[pl.ds(16 * j, 16)] = dumpv

        @pl.when(sid == 0)
        def _prefill_shared():
            pltpu.sync_copy(cv, sh_v.at[pl.ds(0, CAP)])
            pltpu.sync_copy(cv, sh_v.at[pl.ds(CAP, CAP)])

        def sel(j, cnt):
            a = plsc.load_gather(alpha_v, [16 * j + lane])
            msk = a >= thr
            inc = msk.astype(jnp.int32)
            pc = plsc.cumsum(inc)
            tot = jnp.sum(inc, axis=0)
            pos = pc - 1 + cnt
            plsc.store_scatter(cv, [pos], a, mask=msk)
            plsc.store_scatter(ci, [pos], basev + 16 * j + lane, mask=msk)
            return cnt + tot

        cnt = lax.fori_loop(0, TPT // 16, sel, jnp.int32(0))
        cnt_pad = (cnt + 15) & (-16)
        meta_v[pl.ds(0, 16)] = jnp.full((16,), cnt_pad, jnp.int32)
        pltpu.sync_copy(meta_v.at[pl.ds(0, 16)],
                        sh_cnts.at[pl.ds(pl.multiple_of(16 * sid, 16), 16)])

    plsc.subcore_barrier()

    @pl.when(cid == 0)
    def _phase2():
        pltpu.sync_copy(sh_cnts, cnts_v)
        cnt_pad = meta_v[pl.ds(0, 16)][0]
        off = jnp.int32(0)
        total = jnp.int32(0)
        for w in range(NTILES):
            cw = cnts_v[pl.ds(16 * w, 16)][0]
            off = off + jnp.where(w < sid, cw, 0)
            total = total + cw
        meta_v[pl.ds(16, 16)] = jnp.full((16,), off, jnp.int32)
        meta_v[pl.ds(32, 16)] = jnp.full((16,), total, jnp.int32)

        def cp(t, carry):
            s = pl.multiple_of(16 * t, 16)
            d = pl.multiple_of(off + 16 * t, 16)
            pltpu.sync_copy(cv.at[pl.ds(s, 16)], sh_v.at[pl.ds(d, 16)])
            pltpu.sync_copy(ci.at[pl.ds(s, 16)], sh_i.at[pl.ds(d, 16)])
            return carry

        lax.fori_loop(0, cnt_pad // 16, cp, jnp.int32(0))

    plsc.subcore_barrier()

    @pl.when(cid == 0)
    def _phase3():
        cnt_pad = meta_v[pl.ds(0, 16)][0]
        total = meta_v[pl.ds(32, 16)][0]
        pltpu.sync_copy(sh_v, gv)
        pltpu.sync_copy(sh_i, gi)
        ng = total // 16

        def own_loop(o, carry):
            ov = plsc.load_gather(cv, [16 * o + lane])
            oi = plsc.load_gather(ci, [16 * o + lane])
            rankvec = jnp.zeros((16,), jnp.int32)
            for half in range(2):
                vs = [jnp.full((16,), ov[8 * half + j], jnp.float32)
                      for j in range(8)]
                ts = [jnp.full((16,), oi[8 * half + j], jnp.int32)
                      for j in range(8)]

                def g_loop(g, accs):
                    gv16 = plsc.load_gather(gv, [16 * g + lane])
                    gi16 = plsc.load_gather(gi, [16 * g + lane])
                    return tuple(
                        accs[j] + jnp.logical_or(
                            gv16 > vs[j],
                            jnp.logical_and(gv16 == vs[j], gi16 < ts[j])
                        ).astype(jnp.int32)
                        for j in range(8))

                accs = lax.fori_loop(
                    0, ng, g_loop,
                    tuple(jnp.zeros((16,), jnp.int32) for _ in range(8)))
                for j in range(8):
                    rj = jnp.sum(accs[j], axis=0)
                    rankvec = jnp.where(lane == 8 * half + j, rj, rankvec)
            plsc.store_scatter(cr, [16 * o + lane], rankvec)
            return carry

        lax.fori_loop(0, cnt_pad // 16, own_loop, jnp.int32(0))

        def emit(t, carry):
            r16 = plsc.load_gather(cr, [16 * t + lane])
            tsplat = jnp.full((16,), t, jnp.int32)
            ridx = jnp.where(r16 < TOP_M, r16, TOP_M + sid)
            plsc.store_scatter(sidx_v, [tsplat, lane], ridx)
            pltpu.async_copy(
                ci.at[pl.ds(pl.multiple_of(16 * t, 16), 16)],
                sh_oi.at[sidx_v.at[t]], dsem).wait()
            return carry

        lax.fori_loop(0, cnt_pad // 16, emit, jnp.int32(0))

    plsc.subcore_barrier()

    @pl.when(cid == 0)
    def _phase4():
        oi_off = pl.multiple_of(32 * sid, 32)
        pltpu.sync_copy(sh_oi.at[pl.ds(oi_off, 32)], obuf32_v)
        w0 = pltpu.async_copy(obuf32_v, outidx_hbm.at[pl.ds(oi_off, 32)],
                              dsem)
        waits = [w0]
        for chbm, cod_v, out_hbm in (
                (km_hbm, cod0_v, outc0_hbm), (kt_hbm, cod1_v, outc1_hbm),
                (kp_hbm, cod2_v, outc2_hbm)):
            waits.append(pltpu.async_copy(chbm.at[obuf32_v], cod_v, dsem))
        for wgo in waits:
            wgo.wait()
        for cod_v, out_hbm in ((cod0_v, outc0_hbm), (cod1_v, outc1_hbm),
                               (cod2_v, outc2_hbm)):
            pltpu.sync_copy(cod_v, out_hbm.at[pl.ds(oi_off, 32)])


_topk_sc = functools.partial(
    pl.kernel,
    out_type=[jax.ShapeDtypeStruct((TOP_M,), jnp.int32),
              jax.ShapeDtypeStruct((TOP_M,), jnp.int32),
              jax.ShapeDtypeStruct((TOP_M,), jnp.int32),
              jax.ShapeDtypeStruct((TOP_M,), jnp.int32)],
    mesh=plsc.VectorSubcoreMesh(core_axis_name="c", subcore_axis_name="s"),
    compiler_params=pltpu.CompilerParams(needs_layout_passes=False),
    scratch_types=[
        pltpu.VMEM((TPT,), jnp.float32),
        pltpu.VMEM((32,), jnp.int32),
        pltpu.VMEM((32,), jnp.int32),
        pltpu.VMEM((32,), jnp.int32),
        pltpu.VMEM((CAP,), jnp.float32),
        pltpu.VMEM((CAP,), jnp.int32),
        pltpu.VMEM((CAP,), jnp.int32),
        pltpu.VMEM((16,), jnp.float32),
        pltpu.VMEM((NTILES * 16,), jnp.int32),
        pltpu.VMEM((GCAP,), jnp.float32),
        pltpu.VMEM((GCAP,), jnp.int32),
        pltpu.VMEM((48,), jnp.int32),
        pltpu.VMEM((CAP // 16, 16), jnp.int32),
        pltpu.VMEM((32,), jnp.int32),
        pltpu.SemaphoreType.DMA,
        pltpu.VMEM_SHARED((NTILES * 16,), jnp.int32),
        pltpu.VMEM_SHARED((GCAP,), jnp.float32),
        pltpu.VMEM_SHARED((GCAP,), jnp.int32),
        pltpu.VMEM_SHARED((OIDX_CAP,), jnp.int32),
    ])(_topk_body)


def kernel(h0, enc_W1, enc_b1, enc_W2, enc_b2, Wm_W, Wm_b, Wt_W, Wt_b,
           Wp_W, Wp_b, cb_m, cb_t, cb_p, key_W, key_b):
    k0, k1, k2, alpha, thr = _dense_call(
        h0, enc_W1, enc_b1, enc_W2, enc_b2, Wm_W, Wm_b, Wt_W, Wt_b,
        Wp_W, Wp_b, cb_m, cb_t, cb_p, key_W, key_b)
    key_idx, s0, s1, s2 = _topk_sc(alpha, thr.reshape(-1), k0, k1, k2)
    codes = jnp.stack((k0, k1, k2), axis=-1)
    return (codes, key_idx, jnp.stack((s0, s1, s2), axis=-1))


# speedup vs baseline: 1.1026x; 1.0048x over previous
import functools

import jax
import jax.numpy as jnp
from jax import lax
from jax.experimental import pallas as pl
from jax.experimental.pallas import tpu as pltpu
from jax.experimental.pallas import tpu_sc as plsc

N = 16384
IN_DIM = 768
HID = 64
K = 1024
TOP_M = 512
BT = 4096
GRID = N // BT
CAPC = 1024

NTILES = 16
TPT = N // NTILES
CAP = TPT
GCAP = 2048
OIDX_CAP = TOP_M + NTILES
OCOD_CAP = 3 * TOP_M + 3 * NTILES
DUMP_RANK = 1 << 20


def _dense_body(h0_ref, w1_ref, b1_ref, w2_ref, b2_ref,
                wm_ref, bm_ref, wt_ref, bt_ref, wp_ref, bp_ref,
                cbm_ref, cbt_ref, cbp_ref, kw_ref, kb_ref,
                k0_ref, k1_ref, k2_ref, alpha_ref, thr_ref,
                cn_ref, abits_ref):
    i = pl.program_id(0)

    @pl.when(i == 0)
    def _init():
        for c, cb_ref in enumerate((cbm_ref, cbt_ref, cbp_ref)):
            cb = cb_ref[...]
            cn_ref[pl.ds(c * K, K), :] = cb / (
                jnp.sqrt(jnp.sum(cb * cb, axis=-1, keepdims=True)) + 1e-8)

    x = h0_ref[...]
    h1 = jax.nn.relu(jnp.dot(x, w1_ref[...]) + b1_ref[...])
    ht = jax.nn.relu(
        jax.lax.dot_general(w2_ref[...], h1, (((0,), (1,)), ((), ())))
        + b2_ref[...])
    idxk = jax.lax.broadcasted_iota(jnp.int32, (K, 8), 0)
    colk = jax.lax.broadcasted_iota(jnp.int32, (K, 8), 1)
    sqk = idxk * idxk
    wmat = jnp.where(
        colk == 0, idxk >> 2,
        jnp.where(colk == 1, idxk & 3,
                  jnp.where(colk == 2, 1,
                            jnp.where(colk == 3, sqk >> 16,
                                      jnp.where(colk == 4, (sqk >> 8) & 255,
                                                jnp.where(colk == 5, sqk & 255,
                                                          0)))))).astype(
                                                              jnp.float32)
    for c, (w_ref, b_ref, cb_ref) in enumerate((
            (wm_ref, bm_ref, cbm_ref), (wt_ref, bt_ref, cbt_ref),
            (wp_ref, bp_ref, cbp_ref))):
        zt = jax.lax.dot_general(
            w_ref[...], ht, (((0,), (0,)), ((), ()))) + b_ref[...]
        znt = zt / (jnp.sqrt(jnp.sum(zt * zt, axis=0, keepdims=True)) + 1e-8)
        dist = -jax.lax.dot_general(
            cn_ref[pl.ds(c * K, K), :], znt, (((1,), (0,)), ((), ())))
        m = jnp.min(dist, axis=0, keepdims=True)
        eq = (dist == m).astype(jnp.float32)
        sums = jax.lax.dot_general(wmat, eq, (((0,), (0,)), ((), ())))
        sa = 4.0 * sums[0:1, :] + sums[1:2, :]
        cnt = sums[2:3, :]
        sq2 = 65536.0 * sums[3:4, :] + 256.0 * sums[4:5, :] + sums[5:6, :]
        tie2 = (sa - jnp.sqrt(jnp.maximum(2.0 * sq2 - sa * sa, 0.0))) * 0.5
        idxf = jnp.where(cnt > 1.5, tie2, sa)
        (k0_ref, k1_ref, k2_ref)[c][...] = idxf.astype(jnp.int32).reshape(BT)
    trow = jax.lax.dot_general(
        kw_ref[...], ht, (((0,), (0,)), ((), ()))) + kb_ref[...]
    asig = jax.nn.sigmoid(trow)
    alpha_ref[...] = asig.reshape(BT)
    abits_ref[pl.ds(i, 1), :] = jax.lax.bitcast_convert_type(asig, jnp.int32)

    @pl.when(i == GRID - 1)
    def _threshold():
        allbits = abits_ref[...]

        def cond(st):
            lo, hi, clo = st
            return jnp.logical_and(clo > CAPC, hi - lo > 1)

        def body(st):
            lo, hi, clo = st
            mid = lo + (hi - lo) // 2
            c = jnp.sum((allbits >= mid).astype(jnp.int32))
            big = c >= TOP_M
            return (jnp.where(big, mid, lo), jnp.where(big, hi, mid),
                    jnp.where(big, c, clo))

        lo, _, _ = lax.while_loop(
            cond, body,
            (jnp.int32(0), jnp.int32(0x7F800000), jnp.int32(N)))
        thr_ref[...] = jnp.full(
            (1, 128), jax.lax.bitcast_convert_type(lo, jnp.float32),
            jnp.float32)


@functools.partial(jax.jit, static_argnames=("interpret",))
def _dense_call(h0, enc_W1, enc_b1, enc_W2, enc_b2, Wm_W, Wm_b, Wt_W, Wt_b,
                Wp_W, Wp_b, cb_m, cb_t, cb_p, key_W, key_b, interpret=False):
    full2 = lambda r, cdim: pl.BlockSpec((r, cdim), lambda i: (0, 0))
    in_specs = [
        pl.BlockSpec((BT, IN_DIM), lambda i: (i, 0)),
        full2(IN_DIM, HID), full2(1, HID),
        full2(HID, HID), full2(HID, 1),
        full2(HID, HID), full2(HID, 1),
        full2(HID, HID), full2(HID, 1),
        full2(HID, HID), full2(HID, 1),
        full2(K, HID), full2(K, HID), full2(K, HID),
        full2(HID, 1), full2(1, 1),
    ]
    out_specs = [
        pl.BlockSpec((BT,), lambda i: (i,)),
        pl.BlockSpec((BT,), lambda i: (i,)),
        pl.BlockSpec((BT,), lambda i: (i,)),
        pl.BlockSpec((BT,), lambda i: (i,)),
        pl.BlockSpec((1, 128), lambda i: (0, 0)),
    ]
    out_shape = [
        jax.ShapeDtypeStruct((N,), jnp.int32),
        jax.ShapeDtypeStruct((N,), jnp.int32),
        jax.ShapeDtypeStruct((N,), jnp.int32),
        jax.ShapeDtypeStruct((N,), jnp.float32),
        jax.ShapeDtypeStruct((1, 128), jnp.float32),
    ]
    k0, k1, k2, alpha, thr = pl.pallas_call(
        _dense_body,
        grid=(GRID,),
        in_specs=in_specs,
        out_specs=out_specs,
        out_shape=out_shape,
        scratch_shapes=[pltpu.VMEM((3 * K, HID), jnp.float32),
                        pltpu.VMEM((GRID, BT), jnp.int32)],
        interpret=interpret,
    )(h0, enc_W1, enc_b1.reshape(1, HID), enc_W2, enc_b2.reshape(HID, 1),
      Wm_W, Wm_b.reshape(HID, 1), Wt_W, Wt_b.reshape(HID, 1),
      Wp_W, Wp_b.reshape(HID, 1),
      cb_m, cb_t, cb_p, key_W, key_b.reshape(1, 1))
    return k0, k1, k2, alpha, thr


def _topk_body(alpha_hbm, thr_hbm, km_hbm, kt_hbm, kp_hbm,
               outidx_hbm, outc0_hbm, outc1_hbm, outc2_hbm,
               alpha_v, cod0_v, cod1_v, cod2_v, cv, ci, cr, thr_v, cnts_v,
               gv, gi, meta_v, sidx_v, obuf32_v, dsem,
               sh_cnts, sh_v, sh_i, sh_oi):
    cid = lax.axis_index("c")
    sid = lax.axis_index("s")
    lane = lax.iota(jnp.int32, 16)
    base = pl.multiple_of(sid * TPT, TPT)

    @pl.when(cid == 0)
    def _phase1():
        pltpu.sync_copy(alpha_hbm.at[pl.ds(base, TPT)], alpha_v)
        pltpu.sync_copy(thr_hbm.at[pl.ds(0, 16)], thr_v)
        thr = thr_v[...]
        neg1 = jnp.full((16,), -1.0, jnp.float32)
        basev = jnp.full((16,), base, jnp.int32)
        dumpv = jnp.full((16,), DUMP_RANK, jnp.int32)
        for j in range(CAP // 16):
            cv[pl.ds(16 * j, 16)] = neg1
            ci[pl.ds(16 * j, 16)] = basev
            cr[pl.ds(16 * j, 16)] = dumpv

        @pl.when(sid == 0)
        def _prefill_shared():
            pltpu.sync_copy(cv, sh_v.at[pl.ds(0, CAP)])
            pltpu.sync_copy(cv, sh_v.at[pl.ds(CAP, CAP)])

        def sel(j, cnt):
            a = plsc.load_gather(alpha_v, [16 * j + lane])
            msk = a >= thr
            inc = msk.astype(jnp.int32)
            pc = plsc.cumsum(inc)
            tot = jnp.sum(inc, axis=0)
            pos = pc - 1 + cnt
            plsc.store_scatter(cv, [pos], a, mask=msk)
            plsc.store_scatter(ci, [pos], basev + 16 * j + lane, mask=msk)
            return cnt + tot

        cnt = lax.fori_loop(0, TPT // 16, sel, jnp.int32(0))
        cnt_pad = (cnt + 15) & (-16)
        meta_v[pl.ds(0, 16)] = jnp.full((16,), cnt_pad, jnp.int32)
        pltpu.sync_copy(meta_v.at[pl.ds(0, 16)],
                        sh_cnts.at[pl.ds(pl.multiple_of(16 * sid, 16), 16)])

    plsc.subcore_barrier()

    @pl.when(cid == 0)
    def _phase2():
        pltpu.sync_copy(sh_cnts, cnts_v)
        cnt_pad = meta_v[pl.ds(0, 16)][0]
        off = jnp.int32(0)
        total = jnp.int32(0)
        for w in range(NTILES):
            cw = cnts_v[pl.ds(16 * w, 16)][0]
            off = off + jnp.where(w < sid, cw, 0)
            total = total + cw
        meta_v[pl.ds(16, 16)] = jnp.full((16,), off, jnp.int32)
        meta_v[pl.ds(32, 16)] = jnp.full((16,), total, jnp.int32)

        def cp(t, carry):
            s = pl.multiple_of(16 * t, 16)
            d = pl.multiple_of(off + 16 * t, 16)
            pltpu.sync_copy(cv.at[pl.ds(s, 16)], sh_v.at[pl.ds(d, 16)])
            pltpu.sync_copy(ci.at[pl.ds(s, 16)], sh_i.at[pl.ds(d, 16)])
            return carry

        lax.fori_loop(0, cnt_pad // 16, cp, jnp.int32(0))

    plsc.subcore_barrier()

    @pl.when(cid == 0)
    def _phase3():
        cnt_pad = meta_v[pl.ds(0, 16)][0]
        total = meta_v[pl.ds(32, 16)][0]
        pltpu.sync_copy(sh_v, gv)
        pltpu.sync_copy(sh_i, gi)
        ng = total // 16

        def own_loop(o, carry):
            ov = plsc.load_gather(cv, [16 * o + lane])
            oi = plsc.load_gather(ci, [16 * o + lane])
            rankvec = jnp.zeros((16,), jnp.int32)
            for half in range(2):
                vs = [jnp.full((16,), ov[8 * half + j], jnp.float32)
                      for j in range(8)]
                ts = [jnp.full((16,), oi[8 * half + j], jnp.int32)
                      for j in range(8)]

                def g_loop(g, accs):
                    gv16 = plsc.load_gather(gv, [16 * g + lane])
                    gi16 = plsc.load_gather(gi, [16 * g + lane])
                    return tuple(
                        accs[j] + jnp.logical_or(
                            gv16 > vs[j],
                            jnp.logical_and(gv16 == vs[j], gi16 < ts[j])
                        ).astype(jnp.int32)
                        for j in range(8))

                accs = lax.fori_loop(
                    0, ng, g_loop,
                    tuple(jnp.zeros((16,), jnp.int32) for _ in range(8)))
                for j in range(8):
                    rj = jnp.sum(accs[j], axis=0)
                    rankvec = jnp.where(lane == 8 * half + j, rj, rankvec)
            plsc.store_scatter(cr, [16 * o + lane], rankvec)
            return carry

        lax.fori_loop(0, cnt_pad // 16, own_loop, jnp.int32(0))

        def emit(t, carry):
            r16 = plsc.load_gather(cr, [16 * t + lane])
            tsplat = jnp.full((16,), t, jnp.int32)
            ridx = jnp.where(r16 < TOP_M, r16, TOP_M + sid)
            plsc.store_scatter(sidx_v, [tsplat, lane], ridx)
            pltpu.async_copy(
                ci.at[pl.ds(pl.multiple_of(16 * t, 16), 16)],
                sh_oi.at[sidx_v.at[t]], dsem).wait()
            return carry

        lax.fori_loop(0, cnt_pad // 16, emit, jnp.int32(0))

    plsc.subcore_barrier()

    @pl.when(cid == 0)
    def _phase4():
        oi_off = pl.multiple_of(32 * sid, 32)
        pltpu.sync_copy(sh_oi.at[pl.ds(oi_off, 32)], obuf32_v)
        w0 = pltpu.async_copy(obuf32_v, outidx_hbm.at[pl.ds(oi_off, 32)],
                              dsem)
        waits = [w0]
        for chbm, cod_v, out_hbm in (
                (km_hbm, cod0_v, outc0_hbm), (kt_hbm, cod1_v, outc1_hbm),
                (kp_hbm, cod2_v, outc2_hbm)):
            waits.append(pltpu.async_copy(chbm.at[obuf32_v], cod_v, dsem))
        for wgo in waits:
            wgo.wait()
        for cod_v, out_hbm in ((cod0_v, outc0_hbm), (cod1_v, outc1_hbm),
                               (cod2_v, outc2_hbm)):
            pltpu.sync_copy(cod_v, out_hbm.at[pl.ds(oi_off, 32)])


_topk_sc = functools.partial(
    pl.kernel,
    out_type=[jax.ShapeDtypeStruct((TOP_M,), jnp.int32),
              jax.ShapeDtypeStruct((TOP_M,), jnp.int32),
              jax.ShapeDtypeStruct((TOP_M,), jnp.int32),
              jax.ShapeDtypeStruct((TOP_M,), jnp.int32)],
    mesh=plsc.VectorSubcoreMesh(core_axis_name="c", subcore_axis_name="s"),
    compiler_params=pltpu.CompilerParams(needs_layout_passes=False),
    scratch_types=[
        pltpu.VMEM((TPT,), jnp.float32),
        pltpu.VMEM((32,), jnp.int32),
        pltpu.VMEM((32,), jnp.int32),
        pltpu.VMEM((32,), jnp.int32),
        pltpu.VMEM((CAP,), jnp.float32),
        pltpu.VMEM((CAP,), jnp.int32),
        pltpu.VMEM((CAP,), jnp.int32),
        pltpu.VMEM((16,), jnp.float32),
        pltpu.VMEM((NTILES * 16,), jnp.int32),
        pltpu.VMEM((GCAP,), jnp.float32),
        pltpu.VMEM((GCAP,), jnp.int32),
        pltpu.VMEM((48,), jnp.int32),
        pltpu.VMEM((CAP // 16, 16), jnp.int32),
        pltpu.VMEM((32,), jnp.int32),
        pltpu.SemaphoreType.DMA,
        pltpu.VMEM_SHARED((NTILES * 16,), jnp.int32),
        pltpu.VMEM_SHARED((GCAP,), jnp.float32),
        pltpu.VMEM_SHARED((GCAP,), jnp.int32),
        pltpu.VMEM_SHARED((OIDX_CAP,), jnp.int32),
    ])(_topk_body)


def kernel(h0, enc_W1, enc_b1, enc_W2, enc_b2, Wm_W, Wm_b, Wt_W, Wt_b,
           Wp_W, Wp_b, cb_m, cb_t, cb_p, key_W, key_b):
    k0, k1, k2, alpha, thr = _dense_call(
        h0, enc_W1, enc_b1, enc_W2, enc_b2, Wm_W, Wm_b, Wt_W, Wt_b,
        Wp_W, Wp_b, cb_m, cb_t, cb_p, key_W, key_b)
    key_idx, s0, s1, s2 = _topk_sc(alpha, thr.reshape(-1), k0, k1, k2)
    codes = jnp.stack((k0, k1, k2), axis=-1)
    return (codes, key_idx, jnp.stack((s0, s1, s2), axis=-1))
